# baseline (speedup 1.0000x reference)
import jax
import jax.numpy as jnp
from jax import lax
from jax.experimental import pallas as pl
from jax.experimental.pallas import tpu as pltpu
from jax.experimental.pallas import tpu_sc as plsc

_NUM_CHARS = 128
_L = 20
_E = 64
_EP = _E // 2
_WORDS = 256 * 50
_NW = 32
_WPT = _WORDS // _NW
_GROUPS = _WPT // 16
_OUT_D = 3 * _E


def _tree_sum(vs):
    while len(vs) > 1:
        nxt = [vs[i] + vs[i + 1] for i in range(0, len(vs) - 1, 2)]
        if len(vs) % 2:
            nxt.append(vs[-1])
        vs = nxt
    return vs[0]


def _sc_body(sntcs_hbm, w2_hbm, out_hbm, chars_v, w2_v, out_v):
    wid = lax.axis_index("s") * 2 + lax.axis_index("c")
    pltpu.sync_copy(sntcs_hbm.at[pl.ds(wid * (_WPT * _L), _WPT * _L)], chars_v)
    pltpu.sync_copy(w2_hbm, w2_v)

    lane = lax.iota(jnp.int32, 16)
    lane_l = lane * _L
    lane_d = lane * _OUT_D

    def group(g, carry):
        cbase = lane_l + g * (16 * _L)
        c = [plsc.load_gather(chars_v, [cbase + l]) for l in range(_L)]

        m = c[0]
        a = jnp.zeros((16,), jnp.int32)
        for l in range(1, _L):
            gt = c[l] > m
            a = jnp.where(gt, l, a)
            m = jnp.where(gt, c[l], m)
        wl = jnp.maximum(a - 1, 0)
        ends = plsc.load_gather(chars_v, [cbase + wl])

        rows = [jnp.where(wl == 0, 0, c[0])]
        rows += [jnp.where(wl == l, 0, c[l]) for l in range(1, _L - 1)]
        rows.append(ends)
        r32 = [r * _EP for r in rows]

        wbase = lane_d + g * (16 * _OUT_D)

        def jbody(j, carry2):
            first_w = plsc.load_gather(w2_v, [r32[0] + j])
            bow_w = _tree_sum([
                plsc.bitcast(plsc.load_gather(w2_v, [r32[l] + j]), jnp.bfloat16)
                for l in range(1, _L - 1)
            ])
            last_w = plsc.load_gather(w2_v, [r32[_L - 1] + j])
            f_a, f_b = plsc.unpack(
                plsc.bitcast(first_w, jnp.bfloat16), format=plsc.PackFormat.INTERLEAVED)
            s_a, s_b = plsc.unpack(bow_w, format=plsc.PackFormat.INTERLEAVED)
            l_a, l_b = plsc.unpack(
                plsc.bitcast(last_w, jnp.bfloat16), format=plsc.PackFormat.INTERLEAVED)
            j2 = 2 * j
            plsc.store_scatter(out_v, [wbase + j2], f_a)
            plsc.store_scatter(out_v, [wbase + (j2 + 1)], f_b)
            plsc.store_scatter(out_v, [wbase + (j2 + _E)], s_a)
            plsc.store_scatter(out_v, [wbase + (j2 + _E + 1)], s_b)
            plsc.store_scatter(out_v, [wbase + (j2 + 2 * _E)], l_a)
            plsc.store_scatter(out_v, [wbase + (j2 + 2 * _E + 1)], l_b)
            return carry2

        lax.fori_loop(0, _EP, jbody, 0)
        return carry

    lax.fori_loop(0, _GROUPS, group, 0)
    pltpu.sync_copy(out_v, out_hbm.at[pl.ds(wid * (_WPT * _OUT_D), _WPT * _OUT_D)])


def kernel(sntcs, W):
    s_flat = sntcs.reshape(-1).astype(jnp.int32)
    wb = W.T.astype(jnp.bfloat16)
    w2 = jax.lax.bitcast_convert_type(
        wb.reshape(_NUM_CHARS, _EP, 2), jnp.int32).reshape(-1)
    mesh = plsc.VectorSubcoreMesh(core_axis_name="c", subcore_axis_name="s")
    run = pl.kernel(
        _sc_body,
        mesh=mesh,
        compiler_params=pltpu.CompilerParams(needs_layout_passes=False),
        out_type=jax.ShapeDtypeStruct((_WORDS * _OUT_D,), jnp.float32),
        scratch_types=[
            pltpu.VMEM((_WPT * _L,), jnp.int32),
            pltpu.VMEM((_NUM_CHARS * _EP,), jnp.int32),
            pltpu.VMEM((_WPT * _OUT_D,), jnp.float32),
        ],
    )
    out = run(s_flat, w2)
    return out.reshape(256, 50, _OUT_D)


# speedup vs baseline: 1.7270x; 1.7270x over previous
import jax
import jax.numpy as jnp
from jax import lax
from jax.experimental import pallas as pl
from jax.experimental.pallas import tpu as pltpu
from jax.experimental.pallas import tpu_sc as plsc

_NUM_CHARS = 128
_L = 20
_E = 64
_WORDS = 256 * 50
_NW = 32
_WPT = _WORDS // _NW
_GROUPS = _WPT // 16
_OUT_D = 3 * _E


def _tree_sum(vs):
    while len(vs) > 1:
        nxt = [vs[i] + vs[i + 1] for i in range(0, len(vs) - 1, 2)]
        if len(vs) % 2:
            nxt.append(vs[-1])
        vs = nxt
    return vs[0]


def _sc_body(sntcs_hbm, w_hbm, out_hbm, chars_v, w_v, out_v):
    wid = lax.axis_index("s") * 2 + lax.axis_index("c")
    pltpu.sync_copy(sntcs_hbm.at[pl.ds(wid * (_WPT * _L), _WPT * _L)], chars_v)
    pltpu.sync_copy(w_hbm, w_v)

    lane = lax.iota(jnp.int32, 16)
    lane_l = lane * _L
    lane_d = lane * _OUT_D

    def group(g, carry):
        cbase = lane_l + g * (16 * _L)
        c = [plsc.load_gather(chars_v, [cbase + l]) for l in range(_L)]

        m = c[0]
        a = jnp.zeros((16,), jnp.int32)
        for l in range(1, _L):
            gt = c[l] > m
            a = jnp.where(gt, l, a)
            m = jnp.where(gt, c[l], m)
        wl = jnp.maximum(a - 1, 0)
        ends = plsc.load_gather(chars_v, [cbase + wl])

        rows = [jnp.where(wl == 0, 0, c[0])]
        rows += [jnp.where(wl == l, 0, c[l]) for l in range(1, _L - 1)]
        rows.append(ends)

        wbase = lane_d + g * (16 * _OUT_D)

        def ebody(e, carry2):
            e128 = e * _NUM_CHARS
            first = plsc.load_gather(w_v, [rows[0] + e128])
            bow = _tree_sum([
                plsc.load_gather(w_v, [rows[l] + e128]) for l in range(1, _L - 1)
            ])
            lastv = plsc.load_gather(w_v, [rows[_L - 1] + e128])
            plsc.store_scatter(out_v, [wbase + e], first)
            plsc.store_scatter(out_v, [wbase + (e + _E)], bow)
            plsc.store_scatter(out_v, [wbase + (e + 2 * _E)], lastv)
            return carry2

        lax.fori_loop(0, _E, ebody, 0)
        return carry

    lax.fori_loop(0, _GROUPS, group, 0)
    pltpu.sync_copy(out_v, out_hbm.at[pl.ds(wid * (_WPT * _OUT_D), _WPT * _OUT_D)])


def kernel(sntcs, W):
    s_flat = sntcs.reshape(-1).astype(jnp.int32)
    w_flat = W.reshape(-1)
    mesh = plsc.VectorSubcoreMesh(core_axis_name="c", subcore_axis_name="s")
    run = pl.kernel(
        _sc_body,
        mesh=mesh,
        compiler_params=pltpu.CompilerParams(needs_layout_passes=False),
        out_type=jax.ShapeDtypeStruct((_WORDS * _OUT_D,), jnp.float32),
        scratch_types=[
            pltpu.VMEM((_WPT * _L,), jnp.int32),
            pltpu.VMEM((_NUM_CHARS * _E,), jnp.float32),
            pltpu.VMEM((_WPT * _OUT_D,), jnp.float32),
        ],
    )
    out = run(s_flat, w_flat)
    return out.reshape(256, 50, _OUT_D)


# speedup vs baseline: 2.4505x; 1.4189x over previous
import jax
import jax.numpy as jnp
from jax import lax
from jax.experimental import pallas as pl
from jax.experimental.pallas import tpu as pltpu
from jax.experimental.pallas import tpu_sc as plsc

_NUM_CHARS = 128
_L = 20
_E = 64
_WORDS = 256 * 50
_NW = 32
_WPT = _WORDS // _NW
_GROUPS = _WPT // 16
_OUT_D = 3 * _E


def _tree_sum(vs):
    while len(vs) > 1:
        nxt = [vs[i] + vs[i + 1] for i in range(0, len(vs) - 1, 2)]
        if len(vs) % 2:
            nxt.append(vs[-1])
        vs = nxt
    return vs[0]


def _sc_body(sntcs_hbm, wt_hbm, out_hbm, chars_v, wt_v, out_v):
    wid = lax.axis_index("s") * 2 + lax.axis_index("c")
    pltpu.sync_copy(sntcs_hbm.at[pl.ds(wid * (_WPT * _L), _WPT * _L)], chars_v)
    pltpu.sync_copy(wt_hbm, wt_v)

    lane = lax.iota(jnp.int32, 16)
    lane_l = lane * _L
    lane_d = lane * _OUT_D

    def group(g, carry):
        cbase = lane_l + g * (16 * _L)
        c = [plsc.load_gather(chars_v, [cbase + l]) for l in range(_L)]

        m = c[0]
        a = jnp.zeros((16,), jnp.int32)
        for l in range(1, _L):
            gt = c[l] > m
            a = jnp.where(gt, l, a)
            m = jnp.where(gt, c[l], m)
        wl = jnp.maximum(a - 1, 0)
        ends = plsc.load_gather(chars_v, [cbase + wl])

        rows = [jnp.where(wl == 0, 0, c[0])]
        rows += [jnp.where(wl == l, 0, c[l]) for l in range(1, _L - 1)]
        rows.append(ends)
        cl64 = [r * _E for r in rows]

        wbase = lane_d + g * (16 * _OUT_D)

        def ebody(e, carry2):
            e16 = e & 15
            dmm = (e - e16) + ((e16 + lane) & 15)
            first = plsc.load_gather(wt_v, [cl64[0] + dmm])
            bow = _tree_sum([
                plsc.load_gather(wt_v, [cl64[l] + dmm]) for l in range(1, _L - 1)
            ])
            lastv = plsc.load_gather(wt_v, [cl64[_L - 1] + dmm])
            ob = wbase + dmm
            plsc.store_scatter(out_v, [ob], first)
            plsc.store_scatter(out_v, [ob + _E], bow)
            plsc.store_scatter(out_v, [ob + 2 * _E], lastv)
            return carry2

        lax.fori_loop(0, _E, ebody, 0)
        return carry

    lax.fori_loop(0, _GROUPS, group, 0)
    pltpu.sync_copy(out_v, out_hbm.at[pl.ds(wid * (_WPT * _OUT_D), _WPT * _OUT_D)])


def kernel(sntcs, W):
    s_flat = sntcs.reshape(-1).astype(jnp.int32)
    wt_flat = W.T.reshape(-1)
    mesh = plsc.VectorSubcoreMesh(core_axis_name="c", subcore_axis_name="s")
    run = pl.kernel(
        _sc_body,
        mesh=mesh,
        compiler_params=pltpu.CompilerParams(needs_layout_passes=False),
        out_type=jax.ShapeDtypeStruct((_WORDS * _OUT_D,), jnp.float32),
        scratch_types=[
            pltpu.VMEM((_WPT * _L,), jnp.int32),
            pltpu.VMEM((_NUM_CHARS * _E,), jnp.float32),
            pltpu.VMEM((_WPT * _OUT_D,), jnp.float32),
        ],
    )
    out = run(s_flat, wt_flat)
    return out.reshape(256, 50, _OUT_D)


# speedup vs baseline: 2.8045x; 1.1445x over previous
import jax
import jax.numpy as jnp
from jax import lax
from jax.experimental import pallas as pl
from jax.experimental.pallas import tpu as pltpu
from jax.experimental.pallas import tpu_sc as plsc

_NUM_CHARS = 128
_L = 20
_E = 64
_WORDS = 256 * 50
_NW = 32
_WPT = _WORDS // _NW
_SPT = _WPT // 50
_GROUPS = _WPT // 16
_OUT_D = 3 * _E


def _lconst(v):
    return jnp.full((16,), v, jnp.int32)


def _tree_sum(vs):
    while len(vs) > 1:
        nxt = [vs[i] + vs[i + 1] for i in range(0, len(vs) - 1, 2)]
        if len(vs) % 2:
            nxt.append(vs[-1])
        vs = nxt
    return vs[0]


def _sc_body(sntcs_hbm, wt_hbm, out_hbm, chars_v, wt_v, out_v):
    wid = lax.axis_index("s") * 2 + lax.axis_index("c")
    pltpu.sync_copy(sntcs_hbm.at[pl.ds(wid * (_WPT * _L), _WPT * _L)], chars_v)
    pltpu.sync_copy(wt_hbm, wt_v)

    lane = lax.iota(jnp.int32, 16)

    def group(g, carry):
        widx = lane + g * 16
        b_v = widx // 50
        w_v = widx - b_v * 50
        cbase = widx * _L
        c = [plsc.load_gather(chars_v, [cbase + l]) for l in range(_L)]

        m = c[0]
        a = jnp.zeros((16,), jnp.int32)
        for l in range(1, _L):
            gt = c[l] > m
            a = jnp.where(gt, l, a)
            m = jnp.where(gt, c[l], m)
        wl = jnp.maximum(a - 1, 0)
        ends = plsc.load_gather(chars_v, [cbase + wl])

        rows = [jnp.where(wl == 0, 0, c[0])]
        rows += [jnp.where(wl == l, 0, c[l]) for l in range(1, _L - 1)]
        rows.append(ends)
        cl64 = [r * _E for r in rows]

        def ebody(e, carry2):
            e16 = e & 15
            dmm = (e - e16) + ((e16 + lane) & 15)
            first = plsc.load_gather(wt_v, [cl64[0] + dmm])
            bow = _tree_sum([
                plsc.load_gather(wt_v, [cl64[l] + dmm]) for l in range(1, _L - 1)
            ])
            lastv = plsc.load_gather(wt_v, [cl64[_L - 1] + dmm])
            plsc.store_scatter(out_v, [b_v, w_v, dmm], first)
            plsc.store_scatter(out_v, [b_v, w_v, dmm + _E], bow)
            plsc.store_scatter(out_v, [b_v, w_v, dmm + 2 * _E], lastv)
            return carry2

        lax.fori_loop(0, _E, ebody, 0)
        return carry

    lax.fori_loop(0, _GROUPS, group, 0)
    pltpu.sync_copy(out_v, out_hbm.at[pl.ds(wid * _SPT, _SPT)])


def kernel(sntcs, W):
    wt_flat = W.T.reshape(-1)
    mesh = plsc.VectorSubcoreMesh(core_axis_name="c", subcore_axis_name="s")
    run = pl.kernel(
        _sc_body,
        mesh=mesh,
        compiler_params=pltpu.CompilerParams(needs_layout_passes=False),
        out_type=jax.ShapeDtypeStruct((256, 50, _OUT_D), jnp.float32),
        scratch_types=[
            pltpu.VMEM((_WPT * _L,), jnp.int32),
            pltpu.VMEM((_NUM_CHARS * _E,), jnp.float32),
            pltpu.VMEM((_SPT, 50, _OUT_D), jnp.float32),
        ],
    )
    return run(sntcs.reshape(-1).astype(jnp.int32), wt_flat)


# speedup vs baseline: 3.2908x; 1.1734x over previous
import jax
import jax.numpy as jnp
from jax import lax
from jax.experimental import pallas as pl
from jax.experimental.pallas import tpu as pltpu
from jax.experimental.pallas import tpu_sc as plsc

_NUM_CHARS = 128
_L = 20
_E = 64
_EP = _E // 2
_WORDS = 256 * 50
_NW = 32
_WPT = _WORDS // _NW
_SPT = _WPT // 50
_GROUPS = _WPT // 16
_OUT_D = 3 * _E


def _tree_sum(vs):
    while len(vs) > 1:
        nxt = [vs[i] + vs[i + 1] for i in range(0, len(vs) - 1, 2)]
        if len(vs) % 2:
            nxt.append(vs[-1])
        vs = nxt
    return vs[0]


def _sc_body(sntcs_hbm, w2_hbm, out_hbm, chars_v, w2_v, out_v):
    wid = lax.axis_index("s") * 2 + lax.axis_index("c")
    pltpu.sync_copy(sntcs_hbm.at[pl.ds(wid * (_WPT * _L), _WPT * _L)], chars_v)
    pltpu.sync_copy(w2_hbm, w2_v)

    lane = lax.iota(jnp.int32, 16)

    def group(g, carry):
        widx = lane + g * 16
        b_v = widx // 50
        w_v = widx - b_v * 50
        cbase = widx * _L
        c = [plsc.load_gather(chars_v, [cbase + l]) for l in range(_L)]

        m = c[0]
        a = jnp.zeros((16,), jnp.int32)
        for l in range(1, _L):
            gt = c[l] > m
            a = jnp.where(gt, l, a)
            m = jnp.where(gt, c[l], m)
        wl = jnp.maximum(a - 1, 0)
        ends = plsc.load_gather(chars_v, [cbase + wl])

        rows = [jnp.where(wl == 0, 0, c[0])]
        rows += [jnp.where(wl == l, 0, c[l]) for l in range(1, _L - 1)]
        rows.append(ends)
        cl32 = [r * _EP for r in rows]

        def jbody(j, carry2):
            j16 = j & 15
            dmm = (j - j16) + ((j16 + lane) & 15)
            first_w = plsc.load_gather(w2_v, [cl32[0] + dmm])
            bow_w = _tree_sum([
                plsc.bitcast(plsc.load_gather(w2_v, [cl32[l] + dmm]), jnp.bfloat16)
                for l in range(1, _L - 1)
            ])
            last_w = plsc.load_gather(w2_v, [cl32[_L - 1] + dmm])
            f_a, f_b = plsc.unpack(
                plsc.bitcast(first_w, jnp.bfloat16), format=plsc.PackFormat.INTERLEAVED)
            s_a, s_b = plsc.unpack(bow_w, format=plsc.PackFormat.INTERLEAVED)
            l_a, l_b = plsc.unpack(
                plsc.bitcast(last_w, jnp.bfloat16), format=plsc.PackFormat.INTERLEAVED)
            da = dmm * 2
            plsc.store_scatter(out_v, [b_v, w_v, da], f_a)
            plsc.store_scatter(out_v, [b_v, w_v, da + 1], f_b)
            plsc.store_scatter(out_v, [b_v, w_v, da + _E], s_a)
            plsc.store_scatter(out_v, [b_v, w_v, da + (_E + 1)], s_b)
            plsc.store_scatter(out_v, [b_v, w_v, da + 2 * _E], l_a)
            plsc.store_scatter(out_v, [b_v, w_v, da + (2 * _E + 1)], l_b)
            return carry2

        lax.fori_loop(0, _EP, jbody, 0)
        return carry

    lax.fori_loop(0, _GROUPS, group, 0)
    pltpu.sync_copy(out_v, out_hbm.at[pl.ds(wid * _SPT, _SPT)])


def kernel(sntcs, W):
    s_flat = sntcs.reshape(-1).astype(jnp.int32)
    wb = W.T.astype(jnp.bfloat16)
    w2 = jax.lax.bitcast_convert_type(
        wb.reshape(_NUM_CHARS, _EP, 2), jnp.int32).reshape(-1)
    mesh = plsc.VectorSubcoreMesh(core_axis_name="c", subcore_axis_name="s")
    run = pl.kernel(
        _sc_body,
        mesh=mesh,
        compiler_params=pltpu.CompilerParams(needs_layout_passes=False),
        out_type=jax.ShapeDtypeStruct((256, 50, _OUT_D), jnp.float32),
        scratch_types=[
            pltpu.VMEM((_WPT * _L,), jnp.int32),
            pltpu.VMEM((_NUM_CHARS * _EP,), jnp.int32),
            pltpu.VMEM((_SPT, 50, _OUT_D), jnp.float32),
        ],
    )
    return run(s_flat, w2)


# speedup vs baseline: 3.2964x; 1.0017x over previous
import jax
import jax.numpy as jnp
from jax import lax
from jax.experimental import pallas as pl
from jax.experimental.pallas import tpu as pltpu
from jax.experimental.pallas import tpu_sc as plsc

_NUM_CHARS = 128
_L = 20
_E = 64
_EP = _E // 2
_WORDS = 256 * 50
_NW = 32
_WPT = _WORDS // _NW
_SPT = _WPT // 50
_GROUPS = _WPT // 16
_OUT_D = 3 * _E


def _tree_sum(vs):
    while len(vs) > 1:
        nxt = [vs[i] + vs[i + 1] for i in range(0, len(vs) - 1, 2)]
        if len(vs) % 2:
            nxt.append(vs[-1])
        vs = nxt
    return vs[0]


def _sc_body(sntcs_hbm, w2_hbm, out_hbm, chars_v, w2_v, out_v):
    wid = lax.axis_index("s") * 2 + lax.axis_index("c")
    start = wid * (_WPT * _L)
    r0 = pl.multiple_of(jnp.minimum((start >> 7) & ~7, 2000 - 72), 8)
    off = start - r0 * 128
    pltpu.sync_copy(sntcs_hbm.at[pl.ds(r0, 72)], chars_v)
    pltpu.sync_copy(w2_hbm, w2_v)

    lane = lax.iota(jnp.int32, 16)

    def group(g, carry):
        widx = lane + g * 16
        b_v = widx // 50
        w_v = widx - b_v * 50
        cbase = off + widx * _L
        def _cld(loc):
            return plsc.load_gather(chars_v, [loc >> 7, loc & 127])
        c = [_cld(cbase + l) for l in range(_L)]

        m = c[0]
        a = jnp.zeros((16,), jnp.int32)
        for l in range(1, _L):
            gt = c[l] > m
            a = jnp.where(gt, l, a)
            m = jnp.where(gt, c[l], m)
        wl = jnp.maximum(a - 1, 0)
        ends = _cld(cbase + wl)

        rows = [jnp.where(wl == 0, 0, c[0])]
        rows += [jnp.where(wl == l, 0, c[l]) for l in range(1, _L - 1)]
        rows.append(ends)
        cl32 = [r * _EP for r in rows]

        def jbody(j, carry2):
            j16 = j & 15
            dmm = (j - j16) + ((j16 + lane) & 15)
            first_w = plsc.load_gather(w2_v, [cl32[0] + dmm])
            bow_w = _tree_sum([
                plsc.bitcast(plsc.load_gather(w2_v, [cl32[l] + dmm]), jnp.bfloat16)
                for l in range(1, _L - 1)
            ])
            last_w = plsc.load_gather(w2_v, [cl32[_L - 1] + dmm])
            f_a, f_b = plsc.unpack(
                plsc.bitcast(first_w, jnp.bfloat16), format=plsc.PackFormat.INTERLEAVED)
            s_a, s_b = plsc.unpack(bow_w, format=plsc.PackFormat.INTERLEAVED)
            l_a, l_b = plsc.unpack(
                plsc.bitcast(last_w, jnp.bfloat16), format=plsc.PackFormat.INTERLEAVED)
            da = dmm * 2
            plsc.store_scatter(out_v, [b_v, w_v, da], f_a)
            plsc.store_scatter(out_v, [b_v, w_v, da + 1], f_b)
            plsc.store_scatter(out_v, [b_v, w_v, da + _E], s_a)
            plsc.store_scatter(out_v, [b_v, w_v, da + (_E + 1)], s_b)
            plsc.store_scatter(out_v, [b_v, w_v, da + 2 * _E], l_a)
            plsc.store_scatter(out_v, [b_v, w_v, da + (2 * _E + 1)], l_b)
            return carry2

        lax.fori_loop(0, _EP, jbody, 0)
        return carry

    lax.fori_loop(0, _GROUPS, group, 0)
    pltpu.sync_copy(out_v, out_hbm.at[pl.ds(wid * _SPT, _SPT)])


def kernel(sntcs, W):
    s2d = sntcs.reshape(2000, 128).astype(jnp.int32)
    wb = W.T.astype(jnp.bfloat16)
    w2 = jax.lax.bitcast_convert_type(
        wb.reshape(_NUM_CHARS, _EP, 2), jnp.int32).reshape(-1)
    mesh = plsc.VectorSubcoreMesh(core_axis_name="c", subcore_axis_name="s")
    run = pl.kernel(
        _sc_body,
        mesh=mesh,
        compiler_params=pltpu.CompilerParams(needs_layout_passes=False),
        out_type=jax.ShapeDtypeStruct((256, 50, _OUT_D), jnp.float32),
        scratch_types=[
            pltpu.VMEM((72, 128), jnp.int32),
            pltpu.VMEM((_NUM_CHARS * _EP,), jnp.int32),
            pltpu.VMEM((_SPT, 50, _OUT_D), jnp.float32),
        ],
    )
    return run(s2d, w2)


# speedup vs baseline: 3.5323x; 1.0716x over previous
import jax
import jax.numpy as jnp
from jax import lax
from jax.experimental import pallas as pl
from jax.experimental.pallas import tpu as pltpu
from jax.experimental.pallas import tpu_sc as plsc

_NUM_CHARS = 128
_L = 20
_E = 64
_EP = _E // 2
_WORDS = 256 * 50
_NW = 32
_WPT = _WORDS // _NW
_SPT = _WPT // 50
_GROUPS = _WPT // 16
_OUT_D = 3 * _E


def _tree_sum(vs):
    while len(vs) > 1:
        nxt = [vs[i] + vs[i + 1] for i in range(0, len(vs) - 1, 2)]
        if len(vs) % 2:
            nxt.append(vs[-1])
        vs = nxt
    return vs[0]


def _sc_body(sntcs_hbm, w2_hbm, out_hbm, chars_v, w2_v, out_v):
    wid = lax.axis_index("s") * 2 + lax.axis_index("c")
    pltpu.sync_copy(sntcs_hbm.at[pl.ds(wid * (_WPT * _L), _WPT * _L)], chars_v)
    pltpu.sync_copy(w2_hbm, w2_v)

    lane = lax.iota(jnp.int32, 16)

    def group(g, carry):
        widx = lane + g * 16
        b_v = widx // 50
        w_v = widx - b_v * 50
        cbase = widx * _L
        c = [plsc.load_gather(chars_v, [cbase + l]) for l in range(_L)]

        m = c[0]
        a = jnp.zeros((16,), jnp.int32)
        for l in range(1, _L):
            gt = c[l] > m
            a = jnp.where(gt, l, a)
            m = jnp.where(gt, c[l], m)
        wl = jnp.maximum(a - 1, 0)
        ends = plsc.load_gather(chars_v, [cbase + wl])

        rows = [jnp.where(wl == 0, 0, c[0])]
        rows += [jnp.where(wl == l, 0, c[l]) for l in range(1, _L - 1)]
        rows.append(ends)
        cl32 = [r * _EP for r in rows]

        @plsc.parallel_loop(0, _EP, unroll=4)
        def jbody(j):
            j16 = j & 15
            dmm = (j - j16) + ((j16 + lane) & 15)
            first_w = plsc.load_gather(w2_v, [cl32[0] + dmm])
            bow_w = _tree_sum([
                plsc.bitcast(plsc.load_gather(w2_v, [cl32[l] + dmm]), jnp.bfloat16)
                for l in range(1, _L - 1)
            ])
            last_w = plsc.load_gather(w2_v, [cl32[_L - 1] + dmm])
            f_a, f_b = plsc.unpack(
                plsc.bitcast(first_w, jnp.bfloat16), format=plsc.PackFormat.INTERLEAVED)
            s_a, s_b = plsc.unpack(bow_w, format=plsc.PackFormat.INTERLEAVED)
            l_a, l_b = plsc.unpack(
                plsc.bitcast(last_w, jnp.bfloat16), format=plsc.PackFormat.INTERLEAVED)
            da = dmm * 2
            plsc.store_scatter(out_v, [b_v, w_v, da], f_a)
            plsc.store_scatter(out_v, [b_v, w_v, da + 1], f_b)
            plsc.store_scatter(out_v, [b_v, w_v, da + _E], s_a)
            plsc.store_scatter(out_v, [b_v, w_v, da + (_E + 1)], s_b)
            plsc.store_scatter(out_v, [b_v, w_v, da + 2 * _E], l_a)
            plsc.store_scatter(out_v, [b_v, w_v, da + (2 * _E + 1)], l_b)

        return carry

    lax.fori_loop(0, _GROUPS, group, 0)
    pltpu.sync_copy(out_v, out_hbm.at[pl.ds(wid * _SPT, _SPT)])


def kernel(sntcs, W):
    s_flat = sntcs.reshape(-1).astype(jnp.int32)
    wb = W.T.astype(jnp.bfloat16)
    w2 = jax.lax.bitcast_convert_type(
        wb.reshape(_NUM_CHARS, _EP, 2), jnp.int32).reshape(-1)
    mesh = plsc.VectorSubcoreMesh(core_axis_name="c", subcore_axis_name="s")
    run = pl.kernel(
        _sc_body,
        mesh=mesh,
        compiler_params=pltpu.CompilerParams(needs_layout_passes=False),
        out_type=jax.ShapeDtypeStruct((256, 50, _OUT_D), jnp.float32),
        scratch_types=[
            pltpu.VMEM((_WPT * _L,), jnp.int32),
            pltpu.VMEM((_NUM_CHARS * _EP,), jnp.int32),
            pltpu.VMEM((_SPT, 50, _OUT_D), jnp.float32),
        ],
    )
    return run(s_flat, w2)


# speedup vs baseline: 3.6000x; 1.0191x over previous
import jax
import jax.numpy as jnp
from jax import lax
from jax.experimental import pallas as pl
from jax.experimental.pallas import tpu as pltpu
from jax.experimental.pallas import tpu_sc as plsc

_NUM_CHARS = 128
_L = 20
_E = 64
_EP = _E // 2
_WORDS = 256 * 50
_NW = 32
_WPT = _WORDS // _NW
_SPT = _WPT // 50
_GROUPS = _WPT // 16
_OUT_D = 3 * _E


def _tree_sum(vs):
    while len(vs) > 1:
        nxt = [vs[i] + vs[i + 1] for i in range(0, len(vs) - 1, 2)]
        if len(vs) % 2:
            nxt.append(vs[-1])
        vs = nxt
    return vs[0]


def _sc_body(sntcs_hbm, w2_hbm, out_hbm, chars_v, w2_v, out_v):
    wid = lax.axis_index("s") * 2 + lax.axis_index("c")
    pltpu.sync_copy(sntcs_hbm.at[pl.ds(wid * (_WPT * _L), _WPT * _L)], chars_v)
    pltpu.sync_copy(w2_hbm, w2_v)

    lane = lax.iota(jnp.int32, 16)

    def group(g, carry):
        widx = lane + g * 16
        b_v = widx // 50
        w_v = widx - b_v * 50
        cbase = widx * _L
        c = [plsc.load_gather(chars_v, [cbase + l]) for l in range(_L)]

        m = c[0]
        a = jnp.zeros((16,), jnp.int32)
        for l in range(1, _L):
            gt = c[l] > m
            a = jnp.where(gt, l, a)
            m = jnp.where(gt, c[l], m)
        wl = jnp.maximum(a - 1, 0)
        ends = plsc.load_gather(chars_v, [cbase + wl])

        rows = [jnp.where(wl == 0, 0, c[0])]
        rows += [jnp.where(wl == l, 0, c[l]) for l in range(1, _L - 1)]
        rows.append(ends)
        cl32 = [r * _EP for r in rows]

        @plsc.parallel_loop(0, _EP, unroll=8)
        def jbody(j):
            j16 = j & 15
            dmm = (j - j16) + ((j16 + lane) & 15)
            first_w = plsc.load_gather(w2_v, [cl32[0] + dmm])
            bow_w = _tree_sum([
                plsc.bitcast(plsc.load_gather(w2_v, [cl32[l] + dmm]), jnp.bfloat16)
                for l in range(1, _L - 1)
            ])
            last_w = plsc.load_gather(w2_v, [cl32[_L - 1] + dmm])
            f_a, f_b = plsc.unpack(
                plsc.bitcast(first_w, jnp.bfloat16), format=plsc.PackFormat.INTERLEAVED)
            s_a, s_b = plsc.unpack(bow_w, format=plsc.PackFormat.INTERLEAVED)
            l_a, l_b = plsc.unpack(
                plsc.bitcast(last_w, jnp.bfloat16), format=plsc.PackFormat.INTERLEAVED)
            da = dmm * 2
            plsc.store_scatter(out_v, [b_v, w_v, da], f_a)
            plsc.store_scatter(out_v, [b_v, w_v, da + 1], f_b)
            plsc.store_scatter(out_v, [b_v, w_v, da + _E], s_a)
            plsc.store_scatter(out_v, [b_v, w_v, da + (_E + 1)], s_b)
            plsc.store_scatter(out_v, [b_v, w_v, da + 2 * _E], l_a)
            plsc.store_scatter(out_v, [b_v, w_v, da + (2 * _E + 1)], l_b)

        return carry

    lax.fori_loop(0, _GROUPS, group, 0)
    pltpu.sync_copy(out_v, out_hbm.at[pl.ds(wid * _SPT, _SPT)])


def kernel(sntcs, W):
    s_flat = sntcs.reshape(-1).astype(jnp.int32)
    wb = W.T.astype(jnp.bfloat16)
    w2 = jax.lax.bitcast_convert_type(
        wb.reshape(_NUM_CHARS, _EP, 2), jnp.int32).reshape(-1)
    mesh = plsc.VectorSubcoreMesh(core_axis_name="c", subcore_axis_name="s")
    run = pl.kernel(
        _sc_body,
        mesh=mesh,
        compiler_params=pltpu.CompilerParams(needs_layout_passes=False),
        out_type=jax.ShapeDtypeStruct((256, 50, _OUT_D), jnp.float32),
        scratch_types=[
            pltpu.VMEM((_WPT * _L,), jnp.int32),
            pltpu.VMEM((_NUM_CHARS * _EP,), jnp.int32),
            pltpu.VMEM((_SPT, 50, _OUT_D), jnp.float32),
        ],
    )
    return run(s_flat, w2)
